# baseline (speedup 1.0000x reference)
import jax
import jax.numpy as jnp
from jax import lax
from jax.experimental import pallas as pl
from jax.experimental.pallas import tpu as pltpu
from jax.experimental.pallas import tpu_sc as plsc

NC = 2
NS = 16
L = 16
NW = NC * NS

B = 16384
F = 128
ROWS_PER_W = B // NW
CHUNK = 128
HALF = CHUNK // 2
NCHUNK = ROWS_PER_W // CHUNK
NSLOT = 3
GROUPS = CHUNK // L
VIEW_COLS = CHUNK
VROWS_PER_W = ROWS_PER_W // VIEW_COLS


def _mf_body(user_hbm, item_hbm, ut_hbm, it_hbm, w_hbm, b_hbm, out_hbm,
             uidx, iidx,
             ubuf0, ubuf1, ubuf2, vbuf0, vbuf1, vbuf2, outv, wv, bv,
             usem0, usem1, usem2, vsem0, vsem1, vsem2):
    wid = lax.axis_index("s") * NC + lax.axis_index("c")
    base = wid * VROWS_PER_W

    pltpu.sync_copy(user_hbm.at[pl.ds(base, VROWS_PER_W)], uidx)
    pltpu.sync_copy(item_hbm.at[pl.ds(base, VROWS_PER_W)], iidx)
    pltpu.sync_copy(w_hbm, wv)
    pltpu.sync_copy(b_hbm, bv)

    ubufs = (ubuf0, ubuf1, ubuf2)
    vbufs = (vbuf0, vbuf1, vbuf2)
    usems = (usem0, usem1, usem2)
    vsems = (vsem0, vsem1, vsem2)

    def copies(j, slot):
        out = []
        for h in range(2):
            rows = pl.ds(h * HALF, HALF)
            out.append(pltpu.make_async_copy(
                ut_hbm.at[uidx.at[j, rows]], ubufs[slot].at[rows], usems[slot]))
            out.append(pltpu.make_async_copy(
                it_hbm.at[iidx.at[j, rows]], vbufs[slot].at[rows], vsems[slot]))
        return out

    def gather(j, slot):
        for c in copies(j, slot):
            c.start()

    def wait(j, slot):
        for c in copies(j, slot):
            c.wait()

    bias = bv[...]
    wregs = [wv[pl.ds(c * L, L)] for c in range(F // L)]

    for j in range(min(NSLOT - 1, NCHUNK)):
        gather(j, j % NSLOT)

    for j in range(NCHUNK):
        slot = j % NSLOT
        if j + NSLOT - 1 < NCHUNK:
            gather(j + NSLOT - 1, (j + NSLOT - 1) % NSLOT)
        wait(j, slot)
        ub = ubufs[slot]
        vb = vbufs[slot]

        for g in range(GROUPS):
            outv[j, pl.ds(g * L, L)] = bias

        jcol = jnp.full((L,), j, dtype=jnp.int32)

        def r_body(r, carry):
            acc = ub[r, pl.ds(0, L)] * vb[r, pl.ds(0, L)] * wregs[0]
            for c in range(1, F // L):
                cu = ub[r, pl.ds(c * L, L)]
                cv = vb[r, pl.ds(c * L, L)]
                acc = acc + cu * cv * wregs[c]
            plsc.addupdate_scatter(outv, [jcol, jnp.full((L,), r, dtype=jnp.int32)], acc)
            return carry

        lax.fori_loop(0, CHUNK, r_body, 0, unroll=4)

    pltpu.sync_copy(outv, out_hbm.at[pl.ds(base, VROWS_PER_W)])


@jax.jit
def _mf(user2d, item2d, user_table, item_table, w_flat, b16):
    kern = pl.kernel(
        _mf_body,
        out_type=jax.ShapeDtypeStruct((B // VIEW_COLS, VIEW_COLS), jnp.float32),
        mesh=plsc.VectorSubcoreMesh(
            core_axis_name="c", subcore_axis_name="s",
            num_cores=NC, num_subcores=NS),
        scratch_types=[
            pltpu.VMEM((VROWS_PER_W, VIEW_COLS), jnp.int32),
            pltpu.VMEM((VROWS_PER_W, VIEW_COLS), jnp.int32),
            pltpu.VMEM((CHUNK, F), jnp.float32),
            pltpu.VMEM((CHUNK, F), jnp.float32),
            pltpu.VMEM((CHUNK, F), jnp.float32),
            pltpu.VMEM((CHUNK, F), jnp.float32),
            pltpu.VMEM((CHUNK, F), jnp.float32),
            pltpu.VMEM((CHUNK, F), jnp.float32),
            pltpu.VMEM((VROWS_PER_W, VIEW_COLS), jnp.float32),
            pltpu.VMEM((F,), jnp.float32),
            pltpu.VMEM((L,), jnp.float32),
            pltpu.SemaphoreType.DMA,
            pltpu.SemaphoreType.DMA,
            pltpu.SemaphoreType.DMA,
            pltpu.SemaphoreType.DMA,
            pltpu.SemaphoreType.DMA,
            pltpu.SemaphoreType.DMA,
        ],
        compiler_params=pltpu.CompilerParams(needs_layout_passes=False),
    )
    return kern(user2d, item2d, user_table, item_table, w_flat, b16)


def kernel(user, item, user_table, item_table, W, b):
    user2d = user.astype(jnp.int32).reshape(B // VIEW_COLS, VIEW_COLS)
    item2d = item.astype(jnp.int32).reshape(B // VIEW_COLS, VIEW_COLS)
    w_flat = W.reshape(F)
    b16 = jnp.broadcast_to(b.astype(jnp.float32), (L,))
    out = _mf(user2d, item2d, user_table, item_table, w_flat, b16)
    return out.reshape(-1)
